# baseline (speedup 1.0000x reference)
import functools
import jax
import jax.numpy as jnp
from jax import lax
from jax.experimental import pallas as pl
from jax.experimental.pallas import tpu as pltpu
from jax.experimental.pallas import tpu_sc as plsc

_ROWS = 2048
_CP = 1024


def _tc_argmax_body(x_ref, lab_ref):
    x = x_ref[0]
    r, c = x.shape
    bits = lax.bitcast_convert_type(x, jnp.int32)
    key = jnp.where(bits >= 0, bits, bits ^ jnp.int32(0x7FFFFFFF))
    ii = lax.broadcasted_iota(jnp.int32, (r, c), 1)
    packed = (key & jnp.int32(-1024)) | (jnp.int32(1023) - ii)
    km = jnp.max(packed, axis=1)
    idx = jnp.int32(1023) - (km & jnp.int32(1023))
    lab_ref[...] = idx.reshape(lab_ref.shape)


def _make_sc(n_label_rows, total):
    mesh = plsc.VectorSubcoreMesh(
        core_axis_name="c", subcore_axis_name="s", num_cores=2, num_subcores=16
    )

    @functools.partial(
        pl.kernel,
        out_type=jax.ShapeDtypeStruct((16,), jnp.float32),
        mesh=mesh,
        scratch_types=[
            pltpu.VMEM((n_label_rows, 128), jnp.int32),
            pltpu.VMEM((_CP,), jnp.int32),
            pltpu.VMEM((_CP,), jnp.float32),
            pltpu.VMEM((_CP,), jnp.int32),
            pltpu.VMEM((_CP,), jnp.float32),
            pltpu.VMEM((16,), jnp.float32),
        ],
        compiler_params=pltpu.CompilerParams(needs_layout_passes=False),
    )
    def k(lab_hbm, ids_hbm, sp_hbm, out_hbm, lab_v, cnt_v, rel_v, ids_v, sp_v, res_v):
        cid = lax.axis_index("c")
        sid = lax.axis_index("s")
        wid = sid * 2 + cid

        @pl.when(wid == 0)
        def _():
            pltpu.sync_copy(lab_hbm, lab_v)
            pltpu.sync_copy(ids_hbm, ids_v)
            pltpu.sync_copy(sp_hbm, sp_v)

            zero16 = jnp.zeros((16,), jnp.int32)
            zf16 = jnp.zeros((16,), jnp.float32)
            for kk in range(_CP // 16):
                cnt_v[pl.ds(kk * 16, 16)] = zero16
                rel_v[pl.ds(kk * 16, 16)] = zf16

            one16 = jnp.full((16,), 1, jnp.int32)

            def hist_row(j, carry):
                for kk in range(8):
                    lv = lab_v[j, pl.ds(kk * 16, 16)]
                    plsc.addupdate_scatter(cnt_v, [lv], one16)
                return carry

            lax.fori_loop(0, n_label_rows, hist_row, 0)

            for kk in range(_CP // 16):
                idc = ids_v[pl.ds(kk * 16, 16)]
                pv = sp_v[pl.ds(kk * 16, 16)]
                plsc.store_scatter(rel_v, [idc], pv)

            num = jnp.zeros((16,), jnp.float32)
            den = jnp.zeros((16,), jnp.float32)
            inv_total = jnp.float32(1.0 / total)
            for kk in range(_CP // 16):
                cnt = cnt_v[pl.ds(kk * 16, 16)].astype(jnp.float32)
                rel = rel_v[pl.ds(kk * 16, 16)]
                present = cnt > 0.0
                d = rel - cnt * inv_total
                num = num + jnp.where(present, d * d, 0.0)
                den = den + jnp.where(present, 1.0, 0.0)

            numt = jnp.full((16,), jnp.sum(num, axis=0), jnp.float32)
            dent = jnp.full((16,), jnp.sum(den, axis=0), jnp.float32)
            res_v[...] = numt / dent
            pltpu.sync_copy(res_v, out_hbm)

    return k


def kernel(input, src_ids, src_proportions):
    b, s, c = input.shape
    n_label_rows = (b * s) // 128
    labels = pl.pallas_call(
        _tc_argmax_body,
        grid=(b, s // _ROWS),
        in_specs=[pl.BlockSpec((1, _ROWS, c), lambda i, j: (i, j, 0))],
        out_specs=pl.BlockSpec(
            (_ROWS // 128, 128), lambda i, j: (i * (2048 // _ROWS) + j, 0)
        ),
        out_shape=jax.ShapeDtypeStruct((n_label_rows, 128), jnp.int32),
    )(input)
    ids_pad = jnp.pad(src_ids, (0, _CP - c), constant_values=_CP - 1)
    sp_pad = jnp.pad(src_proportions, (0, _CP - c))
    out = _make_sc(n_label_rows, b * s)(labels, ids_pad, sp_pad)
    return out[0]


# speedup vs baseline: 1.0210x; 1.0210x over previous
import functools
import jax
import jax.numpy as jnp
from jax import lax
from jax.experimental import pallas as pl
from jax.experimental.pallas import tpu as pltpu
from jax.experimental.pallas import tpu_sc as plsc

_ROWS = 2048
_CP = 1024


def _tc_argmax_body(x_ref, lab_ref):
    x = x_ref[0]
    r, c = x.shape
    m = jnp.max(x, axis=1, keepdims=True)
    ii = lax.broadcasted_iota(jnp.int32, (r, c), 1)
    idx = jnp.min(jnp.where(x == m, ii, c), axis=1)
    lab_ref[...] = idx.reshape(lab_ref.shape)


def _make_sc(n_label_rows, total):
    mesh = plsc.VectorSubcoreMesh(
        core_axis_name="c", subcore_axis_name="s", num_cores=2, num_subcores=16
    )

    @functools.partial(
        pl.kernel,
        out_type=jax.ShapeDtypeStruct((16,), jnp.float32),
        mesh=mesh,
        scratch_types=[
            pltpu.VMEM((n_label_rows, 128), jnp.int32),
            pltpu.VMEM((_CP,), jnp.int32),
            pltpu.VMEM((_CP,), jnp.float32),
            pltpu.VMEM((_CP,), jnp.int32),
            pltpu.VMEM((_CP,), jnp.float32),
            pltpu.VMEM((16,), jnp.float32),
        ],
        compiler_params=pltpu.CompilerParams(needs_layout_passes=False),
    )
    def k(lab_hbm, ids_hbm, sp_hbm, out_hbm, lab_v, cnt_v, rel_v, ids_v, sp_v, res_v):
        cid = lax.axis_index("c")
        sid = lax.axis_index("s")
        wid = sid * 2 + cid

        @pl.when(wid == 0)
        def _():
            pltpu.sync_copy(lab_hbm, lab_v)
            pltpu.sync_copy(ids_hbm, ids_v)
            pltpu.sync_copy(sp_hbm, sp_v)

            zero16 = jnp.zeros((16,), jnp.int32)
            zf16 = jnp.zeros((16,), jnp.float32)
            for kk in range(_CP // 16):
                cnt_v[pl.ds(kk * 16, 16)] = zero16
                rel_v[pl.ds(kk * 16, 16)] = zf16

            one16 = jnp.full((16,), 1, jnp.int32)

            def hist_row(j, carry):
                for kk in range(8):
                    lv = lab_v[j, pl.ds(kk * 16, 16)]
                    plsc.addupdate_scatter(cnt_v, [lv], one16)
                return carry

            lax.fori_loop(0, n_label_rows, hist_row, 0)

            for kk in range(_CP // 16):
                idc = ids_v[pl.ds(kk * 16, 16)]
                pv = sp_v[pl.ds(kk * 16, 16)]
                plsc.store_scatter(rel_v, [idc], pv)

            num = jnp.zeros((16,), jnp.float32)
            den = jnp.zeros((16,), jnp.float32)
            inv_total = jnp.float32(1.0 / total)
            for kk in range(_CP // 16):
                cnt = cnt_v[pl.ds(kk * 16, 16)].astype(jnp.float32)
                rel = rel_v[pl.ds(kk * 16, 16)]
                present = cnt > 0.0
                d = rel - cnt * inv_total
                num = num + jnp.where(present, d * d, 0.0)
                den = den + jnp.where(present, 1.0, 0.0)

            numt = jnp.full((16,), jnp.sum(num, axis=0), jnp.float32)
            dent = jnp.full((16,), jnp.sum(den, axis=0), jnp.float32)
            res_v[...] = numt / dent
            pltpu.sync_copy(res_v, out_hbm)

    return k


def kernel(input, src_ids, src_proportions):
    b, s, c = input.shape
    n_label_rows = (b * s) // 128
    labels = pl.pallas_call(
        _tc_argmax_body,
        grid=(b, s // _ROWS),
        in_specs=[pl.BlockSpec((1, _ROWS, c), lambda i, j: (i, j, 0))],
        out_specs=pl.BlockSpec(
            (_ROWS // 128, 128), lambda i, j: (i * (2048 // _ROWS) + j, 0)
        ),
        out_shape=jax.ShapeDtypeStruct((n_label_rows, 128), jnp.int32),
    )(input)
    ids_pad = jnp.pad(src_ids, (0, _CP - c), constant_values=_CP - 1)
    sp_pad = jnp.pad(src_proportions, (0, _CP - c))
    out = _make_sc(n_label_rows, b * s)(labels, ids_pad, sp_pad)
    return out[0]


# speedup vs baseline: 1.0514x; 1.0298x over previous
import functools
import jax
import jax.numpy as jnp
from jax import lax
from jax.experimental import pallas as pl
from jax.experimental.pallas import tpu as pltpu
from jax.experimental.pallas import tpu_sc as plsc

_ROWS = 2048
_CP = 1024


def _tc_argmax_body(x_ref, lab_ref):
    x = x_ref[0]
    r, c = x.shape
    m = jnp.max(x, axis=1, keepdims=True)
    ii = lax.broadcasted_iota(jnp.int32, (r, c), 1)
    idx = jnp.min(jnp.where(x == m, ii, c), axis=1)
    lab_ref[...] = idx.reshape(lab_ref.shape)


def _make_sc(n_label_rows, total):
    mesh = plsc.VectorSubcoreMesh(
        core_axis_name="c", subcore_axis_name="s", num_cores=2, num_subcores=16
    )

    rows_per_tec = n_label_rows // 16

    @functools.partial(
        pl.kernel,
        out_type=jax.ShapeDtypeStruct((16,), jnp.float32),
        mesh=mesh,
        scratch_types=[
            pltpu.VMEM((rows_per_tec, 128), jnp.int32),
            pltpu.VMEM((_CP,), jnp.int32),
            pltpu.VMEM((_CP,), jnp.float32),
            pltpu.VMEM((_CP,), jnp.int32),
            pltpu.VMEM((_CP,), jnp.float32),
            pltpu.VMEM((16,), jnp.float32),
            pltpu.VMEM((16, _CP), jnp.int32),
            pltpu.VMEM_SHARED((16, _CP), jnp.int32),
        ],
        compiler_params=pltpu.CompilerParams(needs_layout_passes=False),
    )
    def k(lab_hbm, ids_hbm, sp_hbm, out_hbm,
          lab_v, cnt_v, rel_v, ids_v, sp_v, res_v, allcnt_v, shared):
        cid = lax.axis_index("c")
        sid = lax.axis_index("s")

        @pl.when(cid == 0)
        def _():
            pltpu.sync_copy(
                lab_hbm.at[pl.ds(sid * rows_per_tec, rows_per_tec), :], lab_v
            )
            zero16 = jnp.zeros((16,), jnp.int32)
            for kk in range(_CP // 16):
                cnt_v[pl.ds(kk * 16, 16)] = zero16

            one16 = jnp.full((16,), 1, jnp.int32)
            for j in range(rows_per_tec):
                for kk in range(8):
                    lv = lab_v[j, pl.ds(kk * 16, 16)]
                    plsc.addupdate_scatter(cnt_v, [lv], one16)

            pltpu.sync_copy(cnt_v, shared.at[sid])
            plsc.subcore_barrier()

            @pl.when(sid == 0)
            def _finish():
                pltpu.sync_copy(shared, allcnt_v)
                pltpu.sync_copy(ids_hbm, ids_v)
                pltpu.sync_copy(sp_hbm, sp_v)

                zf16 = jnp.zeros((16,), jnp.float32)
                for kk in range(_CP // 16):
                    rel_v[pl.ds(kk * 16, 16)] = zf16
                for kk in range(_CP // 16):
                    idc = ids_v[pl.ds(kk * 16, 16)]
                    pv = sp_v[pl.ds(kk * 16, 16)]
                    plsc.store_scatter(rel_v, [idc], pv)

                num = jnp.zeros((16,), jnp.float32)
                den = jnp.zeros((16,), jnp.float32)
                inv_total = jnp.float32(1.0 / total)
                for kk in range(_CP // 16):
                    cnt = allcnt_v[0, pl.ds(kk * 16, 16)]
                    for j in range(1, 16):
                        cnt = cnt + allcnt_v[j, pl.ds(kk * 16, 16)]
                    cntf = cnt.astype(jnp.float32)
                    rel = rel_v[pl.ds(kk * 16, 16)]
                    present = cntf > 0.0
                    d = rel - cntf * inv_total
                    num = num + jnp.where(present, d * d, 0.0)
                    den = den + jnp.where(present, 1.0, 0.0)

                numt = jnp.full((16,), jnp.sum(num, axis=0), jnp.float32)
                dent = jnp.full((16,), jnp.sum(den, axis=0), jnp.float32)
                res_v[...] = numt / dent
                pltpu.sync_copy(res_v, out_hbm)

    return k


def kernel(input, src_ids, src_proportions):
    b, s, c = input.shape
    n_label_rows = (b * s) // 128
    labels = pl.pallas_call(
        _tc_argmax_body,
        grid=(b, s // _ROWS),
        in_specs=[pl.BlockSpec((1, _ROWS, c), lambda i, j: (i, j, 0))],
        out_specs=pl.BlockSpec(
            (_ROWS // 128, 128), lambda i, j: (i * (2048 // _ROWS) + j, 0)
        ),
        out_shape=jax.ShapeDtypeStruct((n_label_rows, 128), jnp.int32),
    )(input)
    ids_pad = jnp.pad(src_ids, (0, _CP - c), constant_values=_CP - 1)
    sp_pad = jnp.pad(src_proportions, (0, _CP - c))
    out = _make_sc(n_label_rows, b * s)(labels, ids_pad, sp_pad)
    return out[0]
